# baseline (speedup 1.0000x reference)
import functools

import jax
import jax.numpy as jnp
from jax import lax
from jax.experimental import pallas as pl
from jax.experimental.pallas import tpu as pltpu
from jax.experimental.pallas import tpu_sc as plsc

f32 = jnp.float32
NC = 2
NS = 16
CH = 40
ZB = 104

_HIGH = lax.Precision.HIGHEST


def _dot(a, b, precision=_HIGH):
    return lax.dot_general(a, b, (((1,), (0,)), ((), ())),
                           precision=precision, preferred_element_type=f32)


def _node_dense_body(h_ref, p_ref, wa1h, wa1p, ba1, wa2h, wa2p, ba2,
                     wb1, bb1, wb2, bb2, wc2, bc2,
                     a1_ref, vt_ref, b1_ref, b2_ref, c2_ref):
    h = h_ref[...]
    p = p_ref[...]
    a1_ref[...] = _dot(h, wa1h[...]) + _dot(p, wa1p[...]) + ba1[...]
    vt_ref[...] = _dot(h, wa2h[...]) + _dot(p, wa2p[...]) + ba2[...]
    b1_ref[...] = _dot(h, wb1[...]) + bb1[...]
    b2_ref[...] = _dot(h, wb2[...]) + bb2[...]
    c2_ref[...] = _dot(p, wc2[...]) + bc2[...]


def _node_dense(h, p, WA1, bA1, WA2, bA2, WB1, bB1, WB2, bB2, WC2, bC2):
    n, d = h.shape
    bn = 512
    row_spec = pl.BlockSpec((bn, d), lambda i: (i, 0))
    w_spec = pl.BlockSpec((d, d), lambda i: (0, 0))
    b_spec = pl.BlockSpec((1, d), lambda i: (0, 0))
    return pl.pallas_call(
        _node_dense_body,
        grid=(pl.cdiv(n, bn),),
        in_specs=[row_spec, row_spec] + [w_spec, w_spec, b_spec] * 2
                 + [w_spec, b_spec] * 3,
        out_specs=[row_spec] * 5,
        out_shape=[jax.ShapeDtypeStruct((n, d), f32)] * 5,
    )(h, p, WA1[:d], WA1[d:], bA1.reshape(1, d), WA2[:d], WA2[d:],
      bA2.reshape(1, d), WB1, bB1.reshape(1, d), WB2, bB2.reshape(1, d),
      WC2, bC2.reshape(1, d))


def _edge_dense_body(e_ref, w_ref, b_ref, out_ref):
    e = e_ref[...]
    w = w_ref[...]
    bf16 = jnp.bfloat16
    eh = e.astype(bf16)
    el = (e - eh.astype(f32)).astype(bf16)
    wh = w.astype(bf16)
    wl = (w - wh.astype(f32)).astype(bf16)
    dflt = lax.Precision.DEFAULT
    out_ref[...] = (_dot(eh, wh, dflt) + _dot(eh, wl, dflt)
                    + _dot(el, wh, dflt) + b_ref[...])


def _edge_out_body(e_ref, sig_ref, out_ref):
    sig = sig_ref[...]
    hat = jnp.log(sig / (1.0 - sig))
    out_ref[...] = e_ref[...] + jnp.maximum(hat, 0.0)


def _edge_out(e, hat):
    m, d = e.shape
    bm = 1024
    spec = pl.BlockSpec((bm, d), lambda i: (i, 0))
    return pl.pallas_call(
        _edge_out_body,
        grid=(pl.cdiv(m, bm),),
        in_specs=[spec, spec],
        out_specs=spec,
        out_shape=jax.ShapeDtypeStruct((m, d), f32),
    )(e, hat)


def _edge_dense(e, WB3, bB3):
    m, d = e.shape
    bm = 1024
    return pl.pallas_call(
        _edge_dense_body,
        grid=(pl.cdiv(m, bm),),
        in_specs=[pl.BlockSpec((bm, d), lambda i: (i, 0)),
                  pl.BlockSpec((d, d), lambda i: (0, 0)),
                  pl.BlockSpec((1, d), lambda i: (0, 0))],
        out_specs=pl.BlockSpec((bm, d), lambda i: (i, 0)),
        out_shape=jax.ShapeDtypeStruct((m, d), f32),
    )(e, WB3, bB3.reshape(1, d))


def _stripe(n):
    stripe = (n // NS) // 8 * 8
    tail = n - stripe * NS
    assert stripe % ZB == 0 and tail % 8 == 0 and tail < ZB
    return stripe, tail


def _fill_zb(zb):
    def zloop(j, carry):
        for k in range(zb.shape[1] // 16):
            zb[j, pl.ds(k * 16, 16)] = jnp.zeros((16,), f32)
        return carry
    lax.fori_loop(0, zb.shape[0], zloop, 0)


def _zero_shared(zb, acc, s, n):
    stripe, tail = _stripe(n)
    for q in range(stripe // ZB):
        pltpu.sync_copy(zb, acc.at[pl.ds(s * stripe + q * ZB, ZB)])
    if tail:
        @pl.when(s == 0)
        def _():
            pltpu.sync_copy(zb.at[pl.ds(0, tail)],
                            acc.at[pl.ds(NS * stripe, tail)])


def _dump_shared(acc, out, s, c, n):
    stripe, tail = _stripe(n)
    for q in range(stripe // ZB):
        r = s * stripe + q * ZB
        pltpu.sync_copy(acc.at[pl.ds(r, ZB)], out.at[pl.ds(c * n + r, ZB)])
    if tail:
        @pl.when(s == 0)
        def _():
            pltpu.sync_copy(acc.at[pl.ds(NS * stripe, tail)],
                            out.at[pl.ds(c * n + NS * stripe, tail)])


def _pass_a_body(n_chunks,
                 b3_hbm, b1_hbm, b2_hbm, src_hbm, dst_hbm,
                 hat_hbm, ssp_hbm,
                 idx_s0, idx_d0, b1g0, b2g0, b3v0,
                 idx_s1, idx_d1, b1g1, b2g1, b3v1,
                 sg, zb, acc,
                 sem_i0, sem_i1, sem_n0, sem_n1, sem_o0, sem_o1):
    c = lax.axis_index("c")
    s = lax.axis_index("s")
    n = acc.shape[0]
    _fill_zb(zb)
    _zero_shared(zb, acc, s, n)
    plsc.subcore_barrier()

    tile = c * NS + s
    base = tile * (n_chunks * CH)
    idx_s = (idx_s0, idx_s1)
    idx_d = (idx_d0, idx_d1)
    b1g = (b1g0, b1g1)
    b2g = (b2g0, b2g1)
    b3v = (b3v0, b3v1)
    sem_i = (sem_i0, sem_i1)
    sem_n = (sem_n0, sem_n1)
    sem_o = (sem_o0, sem_o1)

    def issue_idx(i, p):
        eb = base + i * CH
        pltpu.async_copy(src_hbm.at[pl.ds(eb, CH)], idx_s[p], sem_i[p])
        pltpu.async_copy(dst_hbm.at[pl.ds(eb, CH)], idx_d[p], sem_i[p])

    def wait_idx(i, p):
        eb = base + i * CH
        pltpu.make_async_copy(src_hbm.at[pl.ds(eb, CH)], idx_s[p], sem_i[p]).wait()
        pltpu.make_async_copy(dst_hbm.at[pl.ds(eb, CH)], idx_d[p], sem_i[p]).wait()

    def issue_inputs(i, p):
        eb = base + i * CH
        pltpu.async_copy(b1_hbm.at[idx_s[p]], b1g[p], sem_n[p])
        pltpu.async_copy(b2_hbm.at[idx_d[p]], b2g[p], sem_n[p])
        pltpu.async_copy(b3_hbm.at[pl.ds(eb, CH)], b3v[p], sem_n[p])

    def wait_inputs(i, p):
        eb = base + i * CH
        pltpu.make_async_copy(b1_hbm.at[idx_s[p]], b1g[p], sem_n[p]).wait()
        pltpu.make_async_copy(b2_hbm.at[idx_d[p]], b2g[p], sem_n[p]).wait()
        pltpu.make_async_copy(b3_hbm.at[pl.ds(eb, CH)], b3v[p], sem_n[p]).wait()

    def process(i, p):
        def comp(j, inner):
            for k in range(8):
                sl = pl.ds(k * 16, 16)
                hat = b1g[p][j, sl] + b2g[p][j, sl] + b3v[p][j, sl]
                sig_v = 1.0 / (1.0 + jnp.exp(-hat))
                b3v[p][j, sl] = sig_v
                sg[j, sl] = sig_v
            return inner
        lax.fori_loop(0, CH, comp, 0)
        pltpu.sync_copy(sg, acc.at[idx_d[p]], add=True)

    def issue_outputs(i, p):
        eb = base + i * CH
        pltpu.async_copy(b3v[p], hat_hbm.at[pl.ds(eb, CH)], sem_o[p])

    def wait_outputs(i, p):
        eb = base + i * CH
        pltpu.make_async_copy(b3v[p], hat_hbm.at[pl.ds(eb, CH)], sem_o[p]).wait()

    def iteration(i, p, first=False, last=False):
        if not first:
            wait_outputs(i - 2, p)
        wait_idx(i, p)
        issue_inputs(i, p)
        q = 1 - p
        wait_inputs(i - 1, q)
        process(i - 1, q)
        if not last:
            issue_idx(i + 1, q)
        issue_outputs(i - 1, q)

    issue_idx(0, 0)
    wait_idx(0, 0)
    issue_inputs(0, 0)
    issue_idx(1, 1)
    iteration(1, 1, first=True)
    iteration(2, 0)

    def pair(gg, carry):
        i = 3 + 2 * gg
        iteration(i, 1)
        iteration(i + 1, 0)
        return carry
    lax.fori_loop(0, (n_chunks - 4) // 2, pair, 0)

    iteration(n_chunks - 1, 1, last=True)
    wait_outputs(n_chunks - 2, 0)
    wait_inputs(n_chunks - 1, 1)
    process(n_chunks - 1, 1)
    issue_outputs(n_chunks - 1, 1)
    wait_outputs(n_chunks - 1, 1)

    plsc.subcore_barrier()
    _dump_shared(acc, ssp_hbm, s, c, n)


def _pass_a(b3, b1, b2, src, dst, n):
    m, d = b3.shape
    n_chunks = m // (NC * NS * CH)
    assert n_chunks % 2 == 0 and n_chunks >= 6
    mesh = plsc.VectorSubcoreMesh(core_axis_name="c", subcore_axis_name="s",
                                  num_cores=NC, num_subcores=NS)
    out_type = [
        jax.ShapeDtypeStruct((m, d), f32),
        jax.ShapeDtypeStruct((NC * n, d), f32),
    ]
    buf = [
        pltpu.VMEM((CH,), jnp.int32),
        pltpu.VMEM((CH,), jnp.int32),
        pltpu.VMEM((CH, d), f32),
        pltpu.VMEM((CH, d), f32),
        pltpu.VMEM((CH, d), f32),
    ]
    scratch = buf + buf + [
        pltpu.VMEM((CH, d), f32),
        pltpu.VMEM((ZB, d), f32),
        pltpu.VMEM_SHARED((n, d), f32),
    ] + [pltpu.SemaphoreType.DMA] * 6
    kern = pl.kernel(functools.partial(_pass_a_body, n_chunks),
                     out_type=out_type, mesh=mesh, scratch_types=scratch)
    return kern(b3, b1, b2, src, dst)


def _pass_b_body(n_chunks,
                 hat_hbm, vt_hbm, c2_hbm, src_hbm, dst_hbm,
                 svp_hbm, spp_hbm,
                 idx_s0, idx_d0, tg0, sv0,
                 idx_s1, idx_d1, tg1, sv1,
                 zb, acc, sem_p0, sem_p1, sem_g0, sem_g1,
                 sem_d0, sem_d1, sem_s0, sem_s1):
    c = lax.axis_index("c")
    s = lax.axis_index("s")
    n = acc.shape[0]
    tile = c * NS + s
    base = tile * (n_chunks * CH)
    _fill_zb(zb)
    idx_s = (idx_s0, idx_s1)
    idx_d = (idx_d0, idx_d1)
    tg = (tg0, tg1)
    sv = (sv0, sv1)
    sem_p = (sem_p0, sem_p1)
    sem_g = (sem_g0, sem_g1)
    sem_d = (sem_d0, sem_d1)
    sem_s = (sem_s0, sem_s1)

    for tab_hbm, out_hbm in ((vt_hbm, svp_hbm), (c2_hbm, spp_hbm)):
        _zero_shared(zb, acc, s, n)
        plsc.subcore_barrier()

        def issue_pre(i, p):
            eb = base + i * CH
            pltpu.async_copy(src_hbm.at[pl.ds(eb, CH)], idx_s[p], sem_p[p])
            pltpu.async_copy(hat_hbm.at[pl.ds(eb, CH)], sv[p], sem_p[p])

        def wait_pre(i, p):
            eb = base + i * CH
            pltpu.make_async_copy(src_hbm.at[pl.ds(eb, CH)], idx_s[p], sem_p[p]).wait()
            pltpu.make_async_copy(hat_hbm.at[pl.ds(eb, CH)], sv[p], sem_p[p]).wait()

        def issue_idxd(i, p):
            eb = base + i * CH
            pltpu.async_copy(dst_hbm.at[pl.ds(eb, CH)], idx_d[p], sem_d[p])

        def wait_idxd(i, p):
            eb = base + i * CH
            pltpu.make_async_copy(dst_hbm.at[pl.ds(eb, CH)], idx_d[p], sem_d[p]).wait()

        def issue_gather(i, p):
            pltpu.async_copy(tab_hbm.at[idx_s[p]], tg[p], sem_g[p])

        def wait_gather(i, p):
            pltpu.make_async_copy(tab_hbm.at[idx_s[p]], tg[p], sem_g[p]).wait()

        def wait_scatter(i, p):
            pltpu.make_async_copy(tg[p], acc.at[idx_d[p]], sem_s[p]).wait()

        def process(i, p):
            def comp(j, inner):
                for k in range(8):
                    sl = pl.ds(k * 16, 16)
                    tg[p][j, sl] = sv[p][j, sl] * tg[p][j, sl]
                return inner
            lax.fori_loop(0, CH, comp, 0)
            wait_idxd(i, p)
            pltpu.async_copy(tg[p], acc.at[idx_d[p]], sem_s[p], add=True)

        def iteration(i, p, first=False, last=False):
            wait_pre(i, p)
            if not first:
                wait_scatter(i - 2, p)
            issue_gather(i, p)
            issue_idxd(i, p)
            q = 1 - p
            wait_gather(i - 1, q)
            process(i - 1, q)
            if not last:
                issue_pre(i + 1, q)

        issue_pre(0, 0)
        wait_pre(0, 0)
        issue_gather(0, 0)
        issue_idxd(0, 0)
        issue_pre(1, 1)
        iteration(1, 1, first=True)
        iteration(2, 0)

        def pair(gg, carry):
            i = 3 + 2 * gg
            iteration(i, 1)
            iteration(i + 1, 0)
            return carry
        lax.fori_loop(0, (n_chunks - 4) // 2, pair, 0)

        iteration(n_chunks - 1, 1, last=True)
        wait_gather(n_chunks - 1, 1)
        process(n_chunks - 1, 1)
        wait_scatter(n_chunks - 2, 0)
        wait_scatter(n_chunks - 1, 1)

        plsc.subcore_barrier()
        _dump_shared(acc, out_hbm, s, c, n)
        plsc.subcore_barrier()


def _pass_b(hat, vt, c2, src, dst, n):
    m, d = hat.shape
    n_chunks = m // (NC * NS * CH)
    assert n_chunks % 2 == 0 and n_chunks >= 6
    mesh = plsc.VectorSubcoreMesh(core_axis_name="c", subcore_axis_name="s",
                                  num_cores=NC, num_subcores=NS)
    out_type = [
        jax.ShapeDtypeStruct((NC * n, d), f32),
        jax.ShapeDtypeStruct((NC * n, d), f32),
    ]
    buf = [
        pltpu.VMEM((CH,), jnp.int32),
        pltpu.VMEM((CH,), jnp.int32),
        pltpu.VMEM((CH, d), f32),
        pltpu.VMEM((CH, d), f32),
    ]
    scratch = buf + buf + [
        pltpu.VMEM((ZB, d), f32),
        pltpu.VMEM_SHARED((n, d), f32),
    ] + [pltpu.SemaphoreType.DMA] * 8
    kern = pl.kernel(functools.partial(_pass_b_body, n_chunks),
                     out_type=out_type, mesh=mesh, scratch_types=scratch)
    return kern(hat, vt, c2, src, dst)


def _final_body(h_ref, p_ref, a1_ref, ssp_ref, svp_ref, spp_ref,
                wc1_ref, bc1_ref, hout_ref, pout_ref):
    denom = ssp_ref[0] + ssp_ref[1] + 1e-6
    sv = (svp_ref[0] + svp_ref[1]) / denom
    sp = (spp_ref[0] + spp_ref[1]) / denom
    h = h_ref[...]
    p = p_ref[...]
    h_new = a1_ref[...] + sv
    p_new = _dot(p, wc1_ref[...]) + bc1_ref[...] + sp
    hout_ref[...] = h + jnp.maximum(h_new, 0.0)
    pout_ref[...] = p + jnp.tanh(p_new)


def _final(h, p, a1, ssp, svp, spp, WC1, bC1):
    n, d = h.shape
    bn = 512
    row_spec = pl.BlockSpec((bn, d), lambda i: (i, 0))
    part_spec = pl.BlockSpec((NC, bn, d), lambda i: (0, i, 0))
    return pl.pallas_call(
        _final_body,
        grid=(pl.cdiv(n, bn),),
        in_specs=[row_spec, row_spec, row_spec, part_spec, part_spec,
                  part_spec,
                  pl.BlockSpec((d, d), lambda i: (0, 0)),
                  pl.BlockSpec((1, d), lambda i: (0, 0))],
        out_specs=[row_spec, row_spec],
        out_shape=[jax.ShapeDtypeStruct((n, d), f32),
                   jax.ShapeDtypeStruct((n, d), f32)],
    )(h, p, a1, ssp.reshape(NC, n, d), svp.reshape(NC, n, d),
      spp.reshape(NC, n, d), WC1, bC1.reshape(1, d))


def kernel(h, e, p, WA1, bA1, WA2, bA2, WB1, bB1, WB2, bB2, WB3, bB3,
           WC1, bC1, WC2, bC2, edge_index):
    n, d = h.shape
    src = edge_index[0]
    dst = edge_index[1]

    a1, vt, b1, b2, c2 = _node_dense(
        h, p, WA1, bA1, WA2, bA2, WB1, bB1, WB2, bB2, WC2, bC2)
    b3 = _edge_dense(e, WB3, bB3)

    sig, ssp = _pass_a(b3, b1, b2, src, dst, n)
    svp, spp = _pass_b(sig, vt, c2, src, dst, n)
    e_out = _edge_out(e, sig)
    h_out, p_out = _final(h, p, a1, ssp, svp, spp, WC1, bC1)
    return (h_out, e_out, p_out)


# speedup vs baseline: 1.0837x; 1.0837x over previous
import functools

import jax
import jax.numpy as jnp
from jax import lax
from jax.experimental import pallas as pl
from jax.experimental.pallas import tpu as pltpu
from jax.experimental.pallas import tpu_sc as plsc

f32 = jnp.float32
NC = 2
NS = 16
CH = 40
ZB = 104

_HIGH = lax.Precision.HIGHEST


def _dot(a, b, precision=_HIGH):
    return lax.dot_general(a, b, (((1,), (0,)), ((), ())),
                           precision=precision, preferred_element_type=f32)


def _node_dense_body(h_ref, p_ref, wa1h, wa1p, ba1, wa2h, wa2p, ba2,
                     wb1, bb1, wb2, bb2, wc2, bc2,
                     a1_ref, vt_ref, b1_ref, b2_ref, c2_ref):
    h = h_ref[...]
    p = p_ref[...]
    a1_ref[...] = _dot(h, wa1h[...]) + _dot(p, wa1p[...]) + ba1[...]
    vt_ref[...] = _dot(h, wa2h[...]) + _dot(p, wa2p[...]) + ba2[...]
    b1_ref[...] = _dot(h, wb1[...]) + bb1[...]
    b2_ref[...] = _dot(h, wb2[...]) + bb2[...]
    c2_ref[...] = _dot(p, wc2[...]) + bc2[...]


def _node_dense(h, p, WA1, bA1, WA2, bA2, WB1, bB1, WB2, bB2, WC2, bC2):
    n, d = h.shape
    bn = 512
    row_spec = pl.BlockSpec((bn, d), lambda i: (i, 0))
    w_spec = pl.BlockSpec((d, d), lambda i: (0, 0))
    b_spec = pl.BlockSpec((1, d), lambda i: (0, 0))
    return pl.pallas_call(
        _node_dense_body,
        grid=(pl.cdiv(n, bn),),
        in_specs=[row_spec, row_spec] + [w_spec, w_spec, b_spec] * 2
                 + [w_spec, b_spec] * 3,
        out_specs=[row_spec] * 5,
        out_shape=[jax.ShapeDtypeStruct((n, d), f32)] * 5,
    )(h, p, WA1[:d], WA1[d:], bA1.reshape(1, d), WA2[:d], WA2[d:],
      bA2.reshape(1, d), WB1, bB1.reshape(1, d), WB2, bB2.reshape(1, d),
      WC2, bC2.reshape(1, d))


def _edge_dense_body(e_ref, w_ref, b_ref, out_ref):
    e = e_ref[...]
    w = w_ref[...]
    bf16 = jnp.bfloat16
    eh = e.astype(bf16)
    el = (e - eh.astype(f32)).astype(bf16)
    wh = w.astype(bf16)
    wl = (w - wh.astype(f32)).astype(bf16)
    dflt = lax.Precision.DEFAULT
    out_ref[...] = (_dot(eh, wh, dflt) + _dot(eh, wl, dflt)
                    + _dot(el, wh, dflt) + b_ref[...])


def _edge_out_body(e_ref, sig_ref, out_ref):
    out_ref[...] = e_ref[...] + jnp.maximum(sig_ref[...], 0.0)


def _edge_out(e, hat):
    m, d = e.shape
    bm = 1024
    spec = pl.BlockSpec((bm, d), lambda i: (i, 0))
    return pl.pallas_call(
        _edge_out_body,
        grid=(pl.cdiv(m, bm),),
        in_specs=[spec, spec],
        out_specs=spec,
        out_shape=jax.ShapeDtypeStruct((m, d), f32),
    )(e, hat)


def _edge_dense(e, WB3, bB3):
    m, d = e.shape
    bm = 2048
    return pl.pallas_call(
        _edge_dense_body,
        grid=(pl.cdiv(m, bm),),
        in_specs=[pl.BlockSpec((bm, d), lambda i: (i, 0)),
                  pl.BlockSpec((d, d), lambda i: (0, 0)),
                  pl.BlockSpec((1, d), lambda i: (0, 0))],
        out_specs=pl.BlockSpec((bm, d), lambda i: (i, 0)),
        out_shape=jax.ShapeDtypeStruct((m, d), f32),
    )(e, WB3, bB3.reshape(1, d))


def _stripe(n):
    stripe = (n // NS) // 8 * 8
    tail = n - stripe * NS
    assert stripe % ZB == 0 and tail % 8 == 0 and tail < ZB
    return stripe, tail


def _fill_zb(zb):
    def zloop(j, carry):
        for k in range(zb.shape[1] // 16):
            zb[j, pl.ds(k * 16, 16)] = jnp.zeros((16,), f32)
        return carry
    lax.fori_loop(0, zb.shape[0], zloop, 0)


def _zero_shared(zb, acc, s, n):
    stripe, tail = _stripe(n)
    for q in range(stripe // ZB):
        pltpu.sync_copy(zb, acc.at[pl.ds(s * stripe + q * ZB, ZB)])
    if tail:
        @pl.when(s == 0)
        def _():
            pltpu.sync_copy(zb.at[pl.ds(0, tail)],
                            acc.at[pl.ds(NS * stripe, tail)])


def _dump_shared(acc, out, s, c, n):
    stripe, tail = _stripe(n)
    for q in range(stripe // ZB):
        r = s * stripe + q * ZB
        pltpu.sync_copy(acc.at[pl.ds(r, ZB)], out.at[pl.ds(c * n + r, ZB)])
    if tail:
        @pl.when(s == 0)
        def _():
            pltpu.sync_copy(acc.at[pl.ds(NS * stripe, tail)],
                            out.at[pl.ds(c * n + NS * stripe, tail)])


def _pass_a_body(n_chunks,
                 b3_hbm, b1_hbm, b2_hbm, src_hbm, dst_hbm,
                 hat_hbm, ssp_hbm,
                 idx_s0, idx_d0, b1g0, b2g0, b3v0,
                 idx_s1, idx_d1, b1g1, b2g1, b3v1,
                 sg, zb, acc,
                 sem_i0, sem_i1, sem_n0, sem_n1, sem_o0, sem_o1):
    c = lax.axis_index("c")
    s = lax.axis_index("s")
    n = acc.shape[0]
    _fill_zb(zb)
    _zero_shared(zb, acc, s, n)
    plsc.subcore_barrier()

    tile = c * NS + s
    base = tile * (n_chunks * CH)
    idx_s = (idx_s0, idx_s1)
    idx_d = (idx_d0, idx_d1)
    b1g = (b1g0, b1g1)
    b2g = (b2g0, b2g1)
    b3v = (b3v0, b3v1)
    sem_i = (sem_i0, sem_i1)
    sem_n = (sem_n0, sem_n1)
    sem_o = (sem_o0, sem_o1)

    def issue_idx(i, p):
        eb = base + i * CH
        pltpu.async_copy(src_hbm.at[pl.ds(eb, CH)], idx_s[p], sem_i[p])
        pltpu.async_copy(dst_hbm.at[pl.ds(eb, CH)], idx_d[p], sem_i[p])

    def wait_idx(i, p):
        eb = base + i * CH
        pltpu.make_async_copy(src_hbm.at[pl.ds(eb, CH)], idx_s[p], sem_i[p]).wait()
        pltpu.make_async_copy(dst_hbm.at[pl.ds(eb, CH)], idx_d[p], sem_i[p]).wait()

    def issue_inputs(i, p):
        eb = base + i * CH
        pltpu.async_copy(b1_hbm.at[idx_s[p]], b1g[p], sem_n[p])
        pltpu.async_copy(b2_hbm.at[idx_d[p]], b2g[p], sem_n[p])
        pltpu.async_copy(b3_hbm.at[pl.ds(eb, CH)], b3v[p], sem_n[p])

    def wait_inputs(i, p):
        eb = base + i * CH
        pltpu.make_async_copy(b1_hbm.at[idx_s[p]], b1g[p], sem_n[p]).wait()
        pltpu.make_async_copy(b2_hbm.at[idx_d[p]], b2g[p], sem_n[p]).wait()
        pltpu.make_async_copy(b3_hbm.at[pl.ds(eb, CH)], b3v[p], sem_n[p]).wait()

    def process(i, p):
        def comp(j, inner):
            for k in range(8):
                sl = pl.ds(k * 16, 16)
                hat = b1g[p][j, sl] + b2g[p][j, sl] + b3v[p][j, sl]
                b3v[p][j, sl] = hat
                sg[j, sl] = 1.0 / (1.0 + jnp.exp(-hat))
            return inner
        lax.fori_loop(0, CH, comp, 0)
        pltpu.sync_copy(sg, acc.at[idx_d[p]], add=True)

    def issue_outputs(i, p):
        eb = base + i * CH
        pltpu.async_copy(b3v[p], hat_hbm.at[pl.ds(eb, CH)], sem_o[p])

    def wait_outputs(i, p):
        eb = base + i * CH
        pltpu.make_async_copy(b3v[p], hat_hbm.at[pl.ds(eb, CH)], sem_o[p]).wait()

    def iteration(i, p, first=False, last=False):
        if not first:
            wait_outputs(i - 2, p)
        wait_idx(i, p)
        issue_inputs(i, p)
        q = 1 - p
        wait_inputs(i - 1, q)
        process(i - 1, q)
        if not last:
            issue_idx(i + 1, q)
        issue_outputs(i - 1, q)

    issue_idx(0, 0)
    wait_idx(0, 0)
    issue_inputs(0, 0)
    issue_idx(1, 1)
    iteration(1, 1, first=True)
    iteration(2, 0)

    def pair(gg, carry):
        i = 3 + 2 * gg
        iteration(i, 1)
        iteration(i + 1, 0)
        return carry
    lax.fori_loop(0, (n_chunks - 4) // 2, pair, 0)

    iteration(n_chunks - 1, 1, last=True)
    wait_outputs(n_chunks - 2, 0)
    wait_inputs(n_chunks - 1, 1)
    process(n_chunks - 1, 1)
    issue_outputs(n_chunks - 1, 1)
    wait_outputs(n_chunks - 1, 1)

    plsc.subcore_barrier()
    _dump_shared(acc, ssp_hbm, s, c, n)


def _pass_a(b3, b1, b2, src, dst, n):
    m, d = b3.shape
    n_chunks = m // (NC * NS * CH)
    assert n_chunks % 2 == 0 and n_chunks >= 6
    mesh = plsc.VectorSubcoreMesh(core_axis_name="c", subcore_axis_name="s",
                                  num_cores=NC, num_subcores=NS)
    out_type = [
        jax.ShapeDtypeStruct((m, d), f32),
        jax.ShapeDtypeStruct((NC * n, d), f32),
    ]
    buf = [
        pltpu.VMEM((CH,), jnp.int32),
        pltpu.VMEM((CH,), jnp.int32),
        pltpu.VMEM((CH, d), f32),
        pltpu.VMEM((CH, d), f32),
        pltpu.VMEM((CH, d), f32),
    ]
    scratch = buf + buf + [
        pltpu.VMEM((CH, d), f32),
        pltpu.VMEM((ZB, d), f32),
        pltpu.VMEM_SHARED((n, d), f32),
    ] + [pltpu.SemaphoreType.DMA] * 6
    kern = pl.kernel(functools.partial(_pass_a_body, n_chunks),
                     out_type=out_type, mesh=mesh, scratch_types=scratch)
    return kern(b3, b1, b2, src, dst)


def _pass_b_body(n_chunks,
                 hat_hbm, vt_hbm, c2_hbm, src_hbm, dst_hbm,
                 svp_hbm, spp_hbm,
                 idx_s0, idx_d0, tg0, sv0,
                 idx_s1, idx_d1, tg1, sv1,
                 zb, acc, sem_p0, sem_p1, sem_g0, sem_g1,
                 sem_d0, sem_d1, sem_s0, sem_s1):
    c = lax.axis_index("c")
    s = lax.axis_index("s")
    n = acc.shape[0]
    tile = c * NS + s
    base = tile * (n_chunks * CH)
    _fill_zb(zb)
    idx_s = (idx_s0, idx_s1)
    idx_d = (idx_d0, idx_d1)
    tg = (tg0, tg1)
    sv = (sv0, sv1)
    sem_p = (sem_p0, sem_p1)
    sem_g = (sem_g0, sem_g1)
    sem_d = (sem_d0, sem_d1)
    sem_s = (sem_s0, sem_s1)

    for tab_hbm, out_hbm in ((vt_hbm, svp_hbm), (c2_hbm, spp_hbm)):
        _zero_shared(zb, acc, s, n)
        plsc.subcore_barrier()

        def issue_pre(i, p):
            eb = base + i * CH
            pltpu.async_copy(src_hbm.at[pl.ds(eb, CH)], idx_s[p], sem_p[p])
            pltpu.async_copy(hat_hbm.at[pl.ds(eb, CH)], sv[p], sem_p[p])

        def wait_pre(i, p):
            eb = base + i * CH
            pltpu.make_async_copy(src_hbm.at[pl.ds(eb, CH)], idx_s[p], sem_p[p]).wait()
            pltpu.make_async_copy(hat_hbm.at[pl.ds(eb, CH)], sv[p], sem_p[p]).wait()

        def issue_idxd(i, p):
            eb = base + i * CH
            pltpu.async_copy(dst_hbm.at[pl.ds(eb, CH)], idx_d[p], sem_d[p])

        def wait_idxd(i, p):
            eb = base + i * CH
            pltpu.make_async_copy(dst_hbm.at[pl.ds(eb, CH)], idx_d[p], sem_d[p]).wait()

        def issue_gather(i, p):
            pltpu.async_copy(tab_hbm.at[idx_s[p]], tg[p], sem_g[p])

        def wait_gather(i, p):
            pltpu.make_async_copy(tab_hbm.at[idx_s[p]], tg[p], sem_g[p]).wait()

        def wait_scatter(i, p):
            pltpu.make_async_copy(tg[p], acc.at[idx_d[p]], sem_s[p]).wait()

        def process(i, p):
            def comp(j, inner):
                for k in range(8):
                    sl = pl.ds(k * 16, 16)
                    sig = 1.0 / (1.0 + jnp.exp(-sv[p][j, sl]))
                    tg[p][j, sl] = sig * tg[p][j, sl]
                return inner
            lax.fori_loop(0, CH, comp, 0)
            wait_idxd(i, p)
            pltpu.async_copy(tg[p], acc.at[idx_d[p]], sem_s[p], add=True)

        def iteration(i, p, first=False, last=False):
            wait_pre(i, p)
            if not first:
                wait_scatter(i - 2, p)
            issue_gather(i, p)
            issue_idxd(i, p)
            q = 1 - p
            wait_gather(i - 1, q)
            process(i - 1, q)
            if not last:
                issue_pre(i + 1, q)

        issue_pre(0, 0)
        wait_pre(0, 0)
        issue_gather(0, 0)
        issue_idxd(0, 0)
        issue_pre(1, 1)
        iteration(1, 1, first=True)
        iteration(2, 0)

        def pair(gg, carry):
            i = 3 + 2 * gg
            iteration(i, 1)
            iteration(i + 1, 0)
            return carry
        lax.fori_loop(0, (n_chunks - 4) // 2, pair, 0)

        iteration(n_chunks - 1, 1, last=True)
        wait_gather(n_chunks - 1, 1)
        process(n_chunks - 1, 1)
        wait_scatter(n_chunks - 2, 0)
        wait_scatter(n_chunks - 1, 1)

        plsc.subcore_barrier()
        _dump_shared(acc, out_hbm, s, c, n)
        plsc.subcore_barrier()


def _pass_b(hat, vt, c2, src, dst, n):
    m, d = hat.shape
    n_chunks = m // (NC * NS * CH)
    assert n_chunks % 2 == 0 and n_chunks >= 6
    mesh = plsc.VectorSubcoreMesh(core_axis_name="c", subcore_axis_name="s",
                                  num_cores=NC, num_subcores=NS)
    out_type = [
        jax.ShapeDtypeStruct((NC * n, d), f32),
        jax.ShapeDtypeStruct((NC * n, d), f32),
    ]
    buf = [
        pltpu.VMEM((CH,), jnp.int32),
        pltpu.VMEM((CH,), jnp.int32),
        pltpu.VMEM((CH, d), f32),
        pltpu.VMEM((CH, d), f32),
    ]
    scratch = buf + buf + [
        pltpu.VMEM((ZB, d), f32),
        pltpu.VMEM_SHARED((n, d), f32),
    ] + [pltpu.SemaphoreType.DMA] * 8
    kern = pl.kernel(functools.partial(_pass_b_body, n_chunks),
                     out_type=out_type, mesh=mesh, scratch_types=scratch)
    return kern(hat, vt, c2, src, dst)


def _final_body(h_ref, p_ref, a1_ref, ssp_ref, svp_ref, spp_ref,
                wc1_ref, bc1_ref, hout_ref, pout_ref):
    denom = ssp_ref[0] + ssp_ref[1] + 1e-6
    sv = (svp_ref[0] + svp_ref[1]) / denom
    sp = (spp_ref[0] + spp_ref[1]) / denom
    h = h_ref[...]
    p = p_ref[...]
    h_new = a1_ref[...] + sv
    p_new = _dot(p, wc1_ref[...]) + bc1_ref[...] + sp
    hout_ref[...] = h + jnp.maximum(h_new, 0.0)
    pout_ref[...] = p + jnp.tanh(p_new)


def _final(h, p, a1, ssp, svp, spp, WC1, bC1):
    n, d = h.shape
    bn = 512
    row_spec = pl.BlockSpec((bn, d), lambda i: (i, 0))
    part_spec = pl.BlockSpec((NC, bn, d), lambda i: (0, i, 0))
    return pl.pallas_call(
        _final_body,
        grid=(pl.cdiv(n, bn),),
        in_specs=[row_spec, row_spec, row_spec, part_spec, part_spec,
                  part_spec,
                  pl.BlockSpec((d, d), lambda i: (0, 0)),
                  pl.BlockSpec((1, d), lambda i: (0, 0))],
        out_specs=[row_spec, row_spec],
        out_shape=[jax.ShapeDtypeStruct((n, d), f32),
                   jax.ShapeDtypeStruct((n, d), f32)],
    )(h, p, a1, ssp.reshape(NC, n, d), svp.reshape(NC, n, d),
      spp.reshape(NC, n, d), WC1, bC1.reshape(1, d))


def kernel(h, e, p, WA1, bA1, WA2, bA2, WB1, bB1, WB2, bB2, WB3, bB3,
           WC1, bC1, WC2, bC2, edge_index):
    n, d = h.shape
    src = edge_index[0]
    dst = edge_index[1]

    a1, vt, b1, b2, c2 = _node_dense(
        h, p, WA1, bA1, WA2, bA2, WB1, bB1, WB2, bB2, WC2, bC2)
    b3 = _edge_dense(e, WB3, bB3)

    sig, ssp = _pass_a(b3, b1, b2, src, dst, n)
    svp, spp = _pass_b(sig, vt, c2, src, dst, n)
    e_out = _edge_out(e, sig)
    h_out, p_out = _final(h, p, a1, ssp, svp, spp, WC1, bC1)
    return (h_out, e_out, p_out)


# speedup vs baseline: 1.1264x; 1.0393x over previous
import functools

import jax
import jax.numpy as jnp
from jax import lax
from jax.experimental import pallas as pl
from jax.experimental.pallas import tpu as pltpu
from jax.experimental.pallas import tpu_sc as plsc

f32 = jnp.float32
NC = 2
NS = 16
CH = 40
ZB = 104

_HIGH = lax.Precision.HIGHEST


def _dot(a, b, precision=_HIGH):
    return lax.dot_general(a, b, (((1,), (0,)), ((), ())),
                           precision=precision, preferred_element_type=f32)


def _node_dense_body(h_ref, p_ref, wa1h, wa1p, ba1, wa2h, wa2p, ba2,
                     wb1, bb1, wb2, bb2, wc2, bc2,
                     a1_ref, vt_ref, b1_ref, b2_ref, c2_ref):
    h = h_ref[...]
    p = p_ref[...]
    a1_ref[...] = _dot(h, wa1h[...]) + _dot(p, wa1p[...]) + ba1[...]
    vt_ref[...] = _dot(h, wa2h[...]) + _dot(p, wa2p[...]) + ba2[...]
    b1_ref[...] = _dot(h, wb1[...]) + bb1[...]
    b2_ref[...] = _dot(h, wb2[...]) + bb2[...]
    c2_ref[...] = _dot(p, wc2[...]) + bc2[...]


def _node_dense(h, p, WA1, bA1, WA2, bA2, WB1, bB1, WB2, bB2, WC2, bC2):
    n, d = h.shape
    bn = 512
    row_spec = pl.BlockSpec((bn, d), lambda i: (i, 0))
    w_spec = pl.BlockSpec((d, d), lambda i: (0, 0))
    b_spec = pl.BlockSpec((1, d), lambda i: (0, 0))
    return pl.pallas_call(
        _node_dense_body,
        grid=(pl.cdiv(n, bn),),
        in_specs=[row_spec, row_spec] + [w_spec, w_spec, b_spec] * 2
                 + [w_spec, b_spec] * 3,
        out_specs=[row_spec] * 5,
        out_shape=[jax.ShapeDtypeStruct((n, d), f32)] * 5,
    )(h, p, WA1[:d], WA1[d:], bA1.reshape(1, d), WA2[:d], WA2[d:],
      bA2.reshape(1, d), WB1, bB1.reshape(1, d), WB2, bB2.reshape(1, d),
      WC2, bC2.reshape(1, d))


def _edge_dense_body(e_ref, w_ref, b_ref, out_ref):
    e = e_ref[...]
    w = w_ref[...]
    bf16 = jnp.bfloat16
    eh = e.astype(bf16)
    el = (e - eh.astype(f32)).astype(bf16)
    wh = w.astype(bf16)
    wl = (w - wh.astype(f32)).astype(bf16)
    dflt = lax.Precision.DEFAULT
    out_ref[...] = (_dot(eh, wh, dflt) + _dot(eh, wl, dflt)
                    + _dot(el, wh, dflt) + b_ref[...])


def _edge_out_body(e_ref, sig_ref, out_ref):
    out_ref[...] = e_ref[...] + jnp.maximum(sig_ref[...], 0.0)


def _edge_out(e, hat):
    m, d = e.shape
    bm = 1024
    spec = pl.BlockSpec((bm, d), lambda i: (i, 0))
    return pl.pallas_call(
        _edge_out_body,
        grid=(pl.cdiv(m, bm),),
        in_specs=[spec, spec],
        out_specs=spec,
        out_shape=jax.ShapeDtypeStruct((m, d), f32),
    )(e, hat)


def _edge_dense(e, WB3, bB3):
    m, d = e.shape
    bm = 4096
    return pl.pallas_call(
        _edge_dense_body,
        grid=(pl.cdiv(m, bm),),
        in_specs=[pl.BlockSpec((bm, d), lambda i: (i, 0)),
                  pl.BlockSpec((d, d), lambda i: (0, 0)),
                  pl.BlockSpec((1, d), lambda i: (0, 0))],
        out_specs=pl.BlockSpec((bm, d), lambda i: (i, 0)),
        out_shape=jax.ShapeDtypeStruct((m, d), f32),
    )(e, WB3, bB3.reshape(1, d))


def _stripe(n):
    stripe = (n // NS) // 8 * 8
    tail = n - stripe * NS
    assert stripe % ZB == 0 and tail % 8 == 0 and tail < ZB
    return stripe, tail


def _fill_zb(zb):
    def zloop(j, carry):
        for k in range(zb.shape[1] // 16):
            zb[j, pl.ds(k * 16, 16)] = jnp.zeros((16,), f32)
        return carry
    lax.fori_loop(0, zb.shape[0], zloop, 0)


def _zero_shared(zb, acc, s, n):
    stripe, tail = _stripe(n)
    for q in range(stripe // ZB):
        pltpu.sync_copy(zb, acc.at[pl.ds(s * stripe + q * ZB, ZB)])
    if tail:
        @pl.when(s == 0)
        def _():
            pltpu.sync_copy(zb.at[pl.ds(0, tail)],
                            acc.at[pl.ds(NS * stripe, tail)])


def _dump_shared(acc, out, s, c, n):
    stripe, tail = _stripe(n)
    for q in range(stripe // ZB):
        r = s * stripe + q * ZB
        pltpu.sync_copy(acc.at[pl.ds(r, ZB)], out.at[pl.ds(c * n + r, ZB)])
    if tail:
        @pl.when(s == 0)
        def _():
            pltpu.sync_copy(acc.at[pl.ds(NS * stripe, tail)],
                            out.at[pl.ds(c * n + NS * stripe, tail)])


def _pass_a_body(n_chunks,
                 b3_hbm, b1_hbm, b2_hbm, src_hbm, dst_hbm,
                 hat_hbm, ssp_hbm,
                 idx_s0, idx_d0, b1g0, b2g0, b3v0,
                 idx_s1, idx_d1, b1g1, b2g1, b3v1,
                 sg, zb, acc,
                 sem_i0, sem_i1, sem_n0, sem_n1, sem_o0, sem_o1):
    c = lax.axis_index("c")
    s = lax.axis_index("s")
    n = acc.shape[0]
    _fill_zb(zb)
    _zero_shared(zb, acc, s, n)
    plsc.subcore_barrier()

    tile = c * NS + s
    base = tile * (n_chunks * CH)
    idx_s = (idx_s0, idx_s1)
    idx_d = (idx_d0, idx_d1)
    b1g = (b1g0, b1g1)
    b2g = (b2g0, b2g1)
    b3v = (b3v0, b3v1)
    sem_i = (sem_i0, sem_i1)
    sem_n = (sem_n0, sem_n1)
    sem_o = (sem_o0, sem_o1)

    def issue_idx(i, p):
        eb = base + i * CH
        pltpu.async_copy(src_hbm.at[pl.ds(eb, CH)], idx_s[p], sem_i[p])
        pltpu.async_copy(dst_hbm.at[pl.ds(eb, CH)], idx_d[p], sem_i[p])

    def wait_idx(i, p):
        eb = base + i * CH
        pltpu.make_async_copy(src_hbm.at[pl.ds(eb, CH)], idx_s[p], sem_i[p]).wait()
        pltpu.make_async_copy(dst_hbm.at[pl.ds(eb, CH)], idx_d[p], sem_i[p]).wait()

    def issue_inputs(i, p):
        eb = base + i * CH
        pltpu.async_copy(b1_hbm.at[idx_s[p]], b1g[p], sem_n[p])
        pltpu.async_copy(b2_hbm.at[idx_d[p]], b2g[p], sem_n[p])
        pltpu.async_copy(b3_hbm.at[pl.ds(eb, CH)], b3v[p], sem_n[p])

    def wait_inputs(i, p):
        eb = base + i * CH
        pltpu.make_async_copy(b1_hbm.at[idx_s[p]], b1g[p], sem_n[p]).wait()
        pltpu.make_async_copy(b2_hbm.at[idx_d[p]], b2g[p], sem_n[p]).wait()
        pltpu.make_async_copy(b3_hbm.at[pl.ds(eb, CH)], b3v[p], sem_n[p]).wait()

    def process(i, p):
        def comp(j, inner):
            for k in range(8):
                sl = pl.ds(k * 16, 16)
                hat = b1g[p][j, sl] + b2g[p][j, sl] + b3v[p][j, sl]
                b3v[p][j, sl] = hat
                sg[j, sl] = 1.0 / (1.0 + jnp.exp(-hat))
            return inner
        lax.fori_loop(0, CH, comp, 0)
        pltpu.sync_copy(sg, acc.at[idx_d[p]], add=True)

    def issue_outputs(i, p):
        eb = base + i * CH
        pltpu.async_copy(b3v[p], hat_hbm.at[pl.ds(eb, CH)], sem_o[p])

    def wait_outputs(i, p):
        eb = base + i * CH
        pltpu.make_async_copy(b3v[p], hat_hbm.at[pl.ds(eb, CH)], sem_o[p]).wait()

    def iteration(i, p, first=False, last=False):
        if not first:
            wait_outputs(i - 2, p)
        wait_idx(i, p)
        issue_inputs(i, p)
        q = 1 - p
        wait_inputs(i - 1, q)
        process(i - 1, q)
        if not last:
            issue_idx(i + 1, q)
        issue_outputs(i - 1, q)

    issue_idx(0, 0)
    wait_idx(0, 0)
    issue_inputs(0, 0)
    issue_idx(1, 1)
    iteration(1, 1, first=True)
    iteration(2, 0)

    def pair(gg, carry):
        i = 3 + 2 * gg
        iteration(i, 1)
        iteration(i + 1, 0)
        return carry
    lax.fori_loop(0, (n_chunks - 4) // 2, pair, 0)

    iteration(n_chunks - 1, 1, last=True)
    wait_outputs(n_chunks - 2, 0)
    wait_inputs(n_chunks - 1, 1)
    process(n_chunks - 1, 1)
    issue_outputs(n_chunks - 1, 1)
    wait_outputs(n_chunks - 1, 1)

    plsc.subcore_barrier()
    _dump_shared(acc, ssp_hbm, s, c, n)


def _pass_a(b3, b1, b2, src, dst, n):
    m, d = b3.shape
    n_chunks = m // (NC * NS * CH)
    assert n_chunks % 2 == 0 and n_chunks >= 6
    mesh = plsc.VectorSubcoreMesh(core_axis_name="c", subcore_axis_name="s",
                                  num_cores=NC, num_subcores=NS)
    out_type = [
        jax.ShapeDtypeStruct((m, d), f32),
        jax.ShapeDtypeStruct((NC * n, d), f32),
    ]
    buf = [
        pltpu.VMEM((CH,), jnp.int32),
        pltpu.VMEM((CH,), jnp.int32),
        pltpu.VMEM((CH, d), f32),
        pltpu.VMEM((CH, d), f32),
        pltpu.VMEM((CH, d), f32),
    ]
    scratch = buf + buf + [
        pltpu.VMEM((CH, d), f32),
        pltpu.VMEM((ZB, d), f32),
        pltpu.VMEM_SHARED((n, d), f32),
    ] + [pltpu.SemaphoreType.DMA] * 6
    kern = pl.kernel(functools.partial(_pass_a_body, n_chunks),
                     out_type=out_type, mesh=mesh, scratch_types=scratch)
    return kern(b3, b1, b2, src, dst)


def _pass_b_body(n_chunks,
                 hat_hbm, vt_hbm, c2_hbm, src_hbm, dst_hbm,
                 svp_hbm, spp_hbm,
                 idx_s0, idx_d0, tg0, sv0,
                 idx_s1, idx_d1, tg1, sv1,
                 zb, acc, sem_p0, sem_p1, sem_g0, sem_g1,
                 sem_d0, sem_d1, sem_s0, sem_s1):
    c = lax.axis_index("c")
    s = lax.axis_index("s")
    n = acc.shape[0]
    tile = c * NS + s
    base = tile * (n_chunks * CH)
    _fill_zb(zb)
    idx_s = (idx_s0, idx_s1)
    idx_d = (idx_d0, idx_d1)
    tg = (tg0, tg1)
    sv = (sv0, sv1)
    sem_p = (sem_p0, sem_p1)
    sem_g = (sem_g0, sem_g1)
    sem_d = (sem_d0, sem_d1)
    sem_s = (sem_s0, sem_s1)

    for tab_hbm, out_hbm in ((vt_hbm, svp_hbm), (c2_hbm, spp_hbm)):
        _zero_shared(zb, acc, s, n)
        plsc.subcore_barrier()

        def issue_pre(i, p):
            eb = base + i * CH
            pltpu.async_copy(src_hbm.at[pl.ds(eb, CH)], idx_s[p], sem_p[p])
            pltpu.async_copy(hat_hbm.at[pl.ds(eb, CH)], sv[p], sem_p[p])

        def wait_pre(i, p):
            eb = base + i * CH
            pltpu.make_async_copy(src_hbm.at[pl.ds(eb, CH)], idx_s[p], sem_p[p]).wait()
            pltpu.make_async_copy(hat_hbm.at[pl.ds(eb, CH)], sv[p], sem_p[p]).wait()

        def issue_idxd(i, p):
            eb = base + i * CH
            pltpu.async_copy(dst_hbm.at[pl.ds(eb, CH)], idx_d[p], sem_d[p])

        def wait_idxd(i, p):
            eb = base + i * CH
            pltpu.make_async_copy(dst_hbm.at[pl.ds(eb, CH)], idx_d[p], sem_d[p]).wait()

        def issue_gather(i, p):
            pltpu.async_copy(tab_hbm.at[idx_s[p]], tg[p], sem_g[p])

        def wait_gather(i, p):
            pltpu.make_async_copy(tab_hbm.at[idx_s[p]], tg[p], sem_g[p]).wait()

        def wait_scatter(i, p):
            pltpu.make_async_copy(tg[p], acc.at[idx_d[p]], sem_s[p]).wait()

        def process(i, p):
            def comp(j, inner):
                for k in range(8):
                    sl = pl.ds(k * 16, 16)
                    sig = 1.0 / (1.0 + jnp.exp(-sv[p][j, sl]))
                    tg[p][j, sl] = sig * tg[p][j, sl]
                return inner
            lax.fori_loop(0, CH, comp, 0)
            wait_idxd(i, p)
            pltpu.async_copy(tg[p], acc.at[idx_d[p]], sem_s[p], add=True)

        def iteration(i, p, first=False, last=False):
            wait_pre(i, p)
            if not first:
                wait_scatter(i - 2, p)
            issue_gather(i, p)
            issue_idxd(i, p)
            q = 1 - p
            wait_gather(i - 1, q)
            process(i - 1, q)
            if not last:
                issue_pre(i + 1, q)

        issue_pre(0, 0)
        wait_pre(0, 0)
        issue_gather(0, 0)
        issue_idxd(0, 0)
        issue_pre(1, 1)
        iteration(1, 1, first=True)
        iteration(2, 0)

        def pair(gg, carry):
            i = 3 + 2 * gg
            iteration(i, 1)
            iteration(i + 1, 0)
            return carry
        lax.fori_loop(0, (n_chunks - 4) // 2, pair, 0)

        iteration(n_chunks - 1, 1, last=True)
        wait_gather(n_chunks - 1, 1)
        process(n_chunks - 1, 1)
        wait_scatter(n_chunks - 2, 0)
        wait_scatter(n_chunks - 1, 1)

        plsc.subcore_barrier()
        _dump_shared(acc, out_hbm, s, c, n)
        plsc.subcore_barrier()


def _pass_b(hat, vt, c2, src, dst, n):
    m, d = hat.shape
    n_chunks = m // (NC * NS * CH)
    assert n_chunks % 2 == 0 and n_chunks >= 6
    mesh = plsc.VectorSubcoreMesh(core_axis_name="c", subcore_axis_name="s",
                                  num_cores=NC, num_subcores=NS)
    out_type = [
        jax.ShapeDtypeStruct((NC * n, d), f32),
        jax.ShapeDtypeStruct((NC * n, d), f32),
    ]
    buf = [
        pltpu.VMEM((CH,), jnp.int32),
        pltpu.VMEM((CH,), jnp.int32),
        pltpu.VMEM((CH, d), f32),
        pltpu.VMEM((CH, d), f32),
    ]
    scratch = buf + buf + [
        pltpu.VMEM((ZB, d), f32),
        pltpu.VMEM_SHARED((n, d), f32),
    ] + [pltpu.SemaphoreType.DMA] * 8
    kern = pl.kernel(functools.partial(_pass_b_body, n_chunks),
                     out_type=out_type, mesh=mesh, scratch_types=scratch)
    return kern(hat, vt, c2, src, dst)


def _final_body(h_ref, p_ref, a1_ref, ssp_ref, svp_ref, spp_ref,
                wc1_ref, bc1_ref, hout_ref, pout_ref):
    denom = ssp_ref[0] + ssp_ref[1] + 1e-6
    sv = (svp_ref[0] + svp_ref[1]) / denom
    sp = (spp_ref[0] + spp_ref[1]) / denom
    h = h_ref[...]
    p = p_ref[...]
    h_new = a1_ref[...] + sv
    p_new = _dot(p, wc1_ref[...]) + bc1_ref[...] + sp
    hout_ref[...] = h + jnp.maximum(h_new, 0.0)
    pout_ref[...] = p + jnp.tanh(p_new)


def _final(h, p, a1, ssp, svp, spp, WC1, bC1):
    n, d = h.shape
    bn = 512
    row_spec = pl.BlockSpec((bn, d), lambda i: (i, 0))
    part_spec = pl.BlockSpec((NC, bn, d), lambda i: (0, i, 0))
    return pl.pallas_call(
        _final_body,
        grid=(pl.cdiv(n, bn),),
        in_specs=[row_spec, row_spec, row_spec, part_spec, part_spec,
                  part_spec,
                  pl.BlockSpec((d, d), lambda i: (0, 0)),
                  pl.BlockSpec((1, d), lambda i: (0, 0))],
        out_specs=[row_spec, row_spec],
        out_shape=[jax.ShapeDtypeStruct((n, d), f32),
                   jax.ShapeDtypeStruct((n, d), f32)],
    )(h, p, a1, ssp.reshape(NC, n, d), svp.reshape(NC, n, d),
      spp.reshape(NC, n, d), WC1, bC1.reshape(1, d))


def kernel(h, e, p, WA1, bA1, WA2, bA2, WB1, bB1, WB2, bB2, WB3, bB3,
           WC1, bC1, WC2, bC2, edge_index):
    n, d = h.shape
    src = edge_index[0]
    dst = edge_index[1]

    a1, vt, b1, b2, c2 = _node_dense(
        h, p, WA1, bA1, WA2, bA2, WB1, bB1, WB2, bB2, WC2, bC2)
    b3 = _edge_dense(e, WB3, bB3)

    sig, ssp = _pass_a(b3, b1, b2, src, dst, n)
    svp, spp = _pass_b(sig, vt, c2, src, dst, n)
    e_out = _edge_out(e, sig)
    h_out, p_out = _final(h, p, a1, ssp, svp, spp, WC1, bC1)
    return (h_out, e_out, p_out)


# speedup vs baseline: 1.1408x; 1.0129x over previous
import functools

import jax
import jax.numpy as jnp
from jax import lax
from jax.experimental import pallas as pl
from jax.experimental.pallas import tpu as pltpu
from jax.experimental.pallas import tpu_sc as plsc

f32 = jnp.float32
NC = 2
NS = 16
CH = 40
ZB = 104

_HIGH = lax.Precision.HIGHEST


def _dot(a, b, precision=_HIGH):
    return lax.dot_general(a, b, (((1,), (0,)), ((), ())),
                           precision=precision, preferred_element_type=f32)


def _node_dense_body(h_ref, p_ref, wa1h, wa1p, ba1, wa2h, wa2p, ba2,
                     wb1, bb1, wb2, bb2, wc2, bc2,
                     a1_ref, vt_ref, b1_ref, b2_ref, c2_ref):
    h = h_ref[...]
    p = p_ref[...]
    a1_ref[...] = _dot(h, wa1h[...]) + _dot(p, wa1p[...]) + ba1[...]
    vt_ref[...] = _dot(h, wa2h[...]) + _dot(p, wa2p[...]) + ba2[...]
    b1_ref[...] = _dot(h, wb1[...]) + bb1[...]
    b2_ref[...] = _dot(h, wb2[...]) + bb2[...]
    c2_ref[...] = _dot(p, wc2[...]) + bc2[...]


def _node_dense(h, p, WA1, bA1, WA2, bA2, WB1, bB1, WB2, bB2, WC2, bC2):
    n, d = h.shape
    bn = 512
    row_spec = pl.BlockSpec((bn, d), lambda i: (i, 0))
    w_spec = pl.BlockSpec((d, d), lambda i: (0, 0))
    b_spec = pl.BlockSpec((1, d), lambda i: (0, 0))
    return pl.pallas_call(
        _node_dense_body,
        grid=(pl.cdiv(n, bn),),
        in_specs=[row_spec, row_spec] + [w_spec, w_spec, b_spec] * 2
                 + [w_spec, b_spec] * 3,
        out_specs=[row_spec] * 5,
        out_shape=[jax.ShapeDtypeStruct((n, d), f32)] * 5,
    )(h, p, WA1[:d], WA1[d:], bA1.reshape(1, d), WA2[:d], WA2[d:],
      bA2.reshape(1, d), WB1, bB1.reshape(1, d), WB2, bB2.reshape(1, d),
      WC2, bC2.reshape(1, d))


def _edge_dense_body(e_ref, w_ref, b_ref, out_ref):
    e = e_ref[...]
    w = w_ref[...]
    bf16 = jnp.bfloat16
    eh = e.astype(bf16)
    el = (e - eh.astype(f32)).astype(bf16)
    wh = w.astype(bf16)
    wl = (w - wh.astype(f32)).astype(bf16)
    dflt = lax.Precision.DEFAULT
    out_ref[...] = (_dot(eh, wh, dflt) + _dot(eh, wl, dflt)
                    + _dot(el, wh, dflt) + b_ref[...])


def _edge_out_body(e_ref, sig_ref, out_ref):
    out_ref[...] = e_ref[...] + jnp.maximum(sig_ref[...], 0.0)


def _edge_out(e, hat):
    m, d = e.shape
    bm = 2048
    spec = pl.BlockSpec((bm, d), lambda i: (i, 0))
    return pl.pallas_call(
        _edge_out_body,
        grid=(pl.cdiv(m, bm),),
        in_specs=[spec, spec],
        out_specs=spec,
        out_shape=jax.ShapeDtypeStruct((m, d), f32),
    )(e, hat)


def _edge_dense(e, WB3, bB3):
    m, d = e.shape
    bm = 8192
    return pl.pallas_call(
        _edge_dense_body,
        grid=(pl.cdiv(m, bm),),
        in_specs=[pl.BlockSpec((bm, d), lambda i: (i, 0)),
                  pl.BlockSpec((d, d), lambda i: (0, 0)),
                  pl.BlockSpec((1, d), lambda i: (0, 0))],
        out_specs=pl.BlockSpec((bm, d), lambda i: (i, 0)),
        out_shape=jax.ShapeDtypeStruct((m, d), f32),
    )(e, WB3, bB3.reshape(1, d))


def _stripe(n):
    stripe = (n // NS) // 8 * 8
    tail = n - stripe * NS
    assert stripe % ZB == 0 and tail % 8 == 0 and tail < ZB
    return stripe, tail


def _fill_zb(zb):
    def zloop(j, carry):
        for k in range(zb.shape[1] // 16):
            zb[j, pl.ds(k * 16, 16)] = jnp.zeros((16,), f32)
        return carry
    lax.fori_loop(0, zb.shape[0], zloop, 0)


def _zero_shared(zb, acc, s, n):
    stripe, tail = _stripe(n)
    for q in range(stripe // ZB):
        pltpu.sync_copy(zb, acc.at[pl.ds(s * stripe + q * ZB, ZB)])
    if tail:
        @pl.when(s == 0)
        def _():
            pltpu.sync_copy(zb.at[pl.ds(0, tail)],
                            acc.at[pl.ds(NS * stripe, tail)])


def _dump_shared(acc, out, s, c, n):
    stripe, tail = _stripe(n)
    for q in range(stripe // ZB):
        r = s * stripe + q * ZB
        pltpu.sync_copy(acc.at[pl.ds(r, ZB)], out.at[pl.ds(c * n + r, ZB)])
    if tail:
        @pl.when(s == 0)
        def _():
            pltpu.sync_copy(acc.at[pl.ds(NS * stripe, tail)],
                            out.at[pl.ds(c * n + NS * stripe, tail)])


def _pass_a_body(n_chunks,
                 b3_hbm, b1_hbm, b2_hbm, src_hbm, dst_hbm,
                 hat_hbm, ssp_hbm,
                 idx_s0, idx_d0, b1g0, b2g0, b3v0,
                 idx_s1, idx_d1, b1g1, b2g1, b3v1,
                 sg, zb, acc,
                 sem_i0, sem_i1, sem_n0, sem_n1, sem_o0, sem_o1):
    c = lax.axis_index("c")
    s = lax.axis_index("s")
    n = acc.shape[0]
    _fill_zb(zb)
    _zero_shared(zb, acc, s, n)
    plsc.subcore_barrier()

    tile = c * NS + s
    base = tile * (n_chunks * CH)
    idx_s = (idx_s0, idx_s1)
    idx_d = (idx_d0, idx_d1)
    b1g = (b1g0, b1g1)
    b2g = (b2g0, b2g1)
    b3v = (b3v0, b3v1)
    sem_i = (sem_i0, sem_i1)
    sem_n = (sem_n0, sem_n1)
    sem_o = (sem_o0, sem_o1)

    def issue_idx(i, p):
        eb = base + i * CH
        pltpu.async_copy(src_hbm.at[pl.ds(eb, CH)], idx_s[p], sem_i[p])
        pltpu.async_copy(dst_hbm.at[pl.ds(eb, CH)], idx_d[p], sem_i[p])

    def wait_idx(i, p):
        eb = base + i * CH
        pltpu.make_async_copy(src_hbm.at[pl.ds(eb, CH)], idx_s[p], sem_i[p]).wait()
        pltpu.make_async_copy(dst_hbm.at[pl.ds(eb, CH)], idx_d[p], sem_i[p]).wait()

    def issue_inputs(i, p):
        eb = base + i * CH
        pltpu.async_copy(b1_hbm.at[idx_s[p]], b1g[p], sem_n[p])
        pltpu.async_copy(b2_hbm.at[idx_d[p]], b2g[p], sem_n[p])
        pltpu.async_copy(b3_hbm.at[pl.ds(eb, CH)], b3v[p], sem_n[p])

    def wait_inputs(i, p):
        eb = base + i * CH
        pltpu.make_async_copy(b1_hbm.at[idx_s[p]], b1g[p], sem_n[p]).wait()
        pltpu.make_async_copy(b2_hbm.at[idx_d[p]], b2g[p], sem_n[p]).wait()
        pltpu.make_async_copy(b3_hbm.at[pl.ds(eb, CH)], b3v[p], sem_n[p]).wait()

    def process(i, p):
        def comp(j, inner):
            for k in range(8):
                sl = pl.ds(k * 16, 16)
                hat = b1g[p][j, sl] + b2g[p][j, sl] + b3v[p][j, sl]
                b3v[p][j, sl] = hat
                sg[j, sl] = 1.0 / (1.0 + jnp.exp(-hat))
            return inner
        lax.fori_loop(0, CH, comp, 0)
        pltpu.sync_copy(sg, acc.at[idx_d[p]], add=True)

    def issue_outputs(i, p):
        eb = base + i * CH
        pltpu.async_copy(b3v[p], hat_hbm.at[pl.ds(eb, CH)], sem_o[p])

    def wait_outputs(i, p):
        eb = base + i * CH
        pltpu.make_async_copy(b3v[p], hat_hbm.at[pl.ds(eb, CH)], sem_o[p]).wait()

    def iteration(i, p, first=False, last=False):
        if not first:
            wait_outputs(i - 2, p)
        wait_idx(i, p)
        issue_inputs(i, p)
        q = 1 - p
        wait_inputs(i - 1, q)
        process(i - 1, q)
        if not last:
            issue_idx(i + 1, q)
        issue_outputs(i - 1, q)

    issue_idx(0, 0)
    wait_idx(0, 0)
    issue_inputs(0, 0)
    issue_idx(1, 1)
    iteration(1, 1, first=True)
    iteration(2, 0)

    def pair(gg, carry):
        i = 3 + 2 * gg
        iteration(i, 1)
        iteration(i + 1, 0)
        return carry
    lax.fori_loop(0, (n_chunks - 4) // 2, pair, 0)

    iteration(n_chunks - 1, 1, last=True)
    wait_outputs(n_chunks - 2, 0)
    wait_inputs(n_chunks - 1, 1)
    process(n_chunks - 1, 1)
    issue_outputs(n_chunks - 1, 1)
    wait_outputs(n_chunks - 1, 1)

    plsc.subcore_barrier()
    _dump_shared(acc, ssp_hbm, s, c, n)


def _pass_a(b3, b1, b2, src, dst, n):
    m, d = b3.shape
    n_chunks = m // (NC * NS * CH)
    assert n_chunks % 2 == 0 and n_chunks >= 6
    mesh = plsc.VectorSubcoreMesh(core_axis_name="c", subcore_axis_name="s",
                                  num_cores=NC, num_subcores=NS)
    out_type = [
        jax.ShapeDtypeStruct((m, d), f32),
        jax.ShapeDtypeStruct((NC * n, d), f32),
    ]
    buf = [
        pltpu.VMEM((CH,), jnp.int32),
        pltpu.VMEM((CH,), jnp.int32),
        pltpu.VMEM((CH, d), f32),
        pltpu.VMEM((CH, d), f32),
        pltpu.VMEM((CH, d), f32),
    ]
    scratch = buf + buf + [
        pltpu.VMEM((CH, d), f32),
        pltpu.VMEM((ZB, d), f32),
        pltpu.VMEM_SHARED((n, d), f32),
    ] + [pltpu.SemaphoreType.DMA] * 6
    kern = pl.kernel(functools.partial(_pass_a_body, n_chunks),
                     out_type=out_type, mesh=mesh, scratch_types=scratch)
    return kern(b3, b1, b2, src, dst)


def _pass_b_body(n_chunks,
                 hat_hbm, vt_hbm, c2_hbm, src_hbm, dst_hbm,
                 svp_hbm, spp_hbm,
                 idx_s0, idx_d0, tg0, sv0,
                 idx_s1, idx_d1, tg1, sv1,
                 zb, acc, sem_p0, sem_p1, sem_g0, sem_g1,
                 sem_d0, sem_d1, sem_s0, sem_s1):
    c = lax.axis_index("c")
    s = lax.axis_index("s")
    n = acc.shape[0]
    tile = c * NS + s
    base = tile * (n_chunks * CH)
    _fill_zb(zb)
    idx_s = (idx_s0, idx_s1)
    idx_d = (idx_d0, idx_d1)
    tg = (tg0, tg1)
    sv = (sv0, sv1)
    sem_p = (sem_p0, sem_p1)
    sem_g = (sem_g0, sem_g1)
    sem_d = (sem_d0, sem_d1)
    sem_s = (sem_s0, sem_s1)

    for tab_hbm, out_hbm in ((vt_hbm, svp_hbm), (c2_hbm, spp_hbm)):
        _zero_shared(zb, acc, s, n)
        plsc.subcore_barrier()

        def issue_pre(i, p):
            eb = base + i * CH
            pltpu.async_copy(src_hbm.at[pl.ds(eb, CH)], idx_s[p], sem_p[p])
            pltpu.async_copy(hat_hbm.at[pl.ds(eb, CH)], sv[p], sem_p[p])

        def wait_pre(i, p):
            eb = base + i * CH
            pltpu.make_async_copy(src_hbm.at[pl.ds(eb, CH)], idx_s[p], sem_p[p]).wait()
            pltpu.make_async_copy(hat_hbm.at[pl.ds(eb, CH)], sv[p], sem_p[p]).wait()

        def issue_idxd(i, p):
            eb = base + i * CH
            pltpu.async_copy(dst_hbm.at[pl.ds(eb, CH)], idx_d[p], sem_d[p])

        def wait_idxd(i, p):
            eb = base + i * CH
            pltpu.make_async_copy(dst_hbm.at[pl.ds(eb, CH)], idx_d[p], sem_d[p]).wait()

        def issue_gather(i, p):
            pltpu.async_copy(tab_hbm.at[idx_s[p]], tg[p], sem_g[p])

        def wait_gather(i, p):
            pltpu.make_async_copy(tab_hbm.at[idx_s[p]], tg[p], sem_g[p]).wait()

        def wait_scatter(i, p):
            pltpu.make_async_copy(tg[p], acc.at[idx_d[p]], sem_s[p]).wait()

        def process(i, p):
            def comp(j, inner):
                for k in range(8):
                    sl = pl.ds(k * 16, 16)
                    sig = 1.0 / (1.0 + jnp.exp(-sv[p][j, sl]))
                    tg[p][j, sl] = sig * tg[p][j, sl]
                return inner
            lax.fori_loop(0, CH, comp, 0)
            wait_idxd(i, p)
            pltpu.async_copy(tg[p], acc.at[idx_d[p]], sem_s[p], add=True)

        def iteration(i, p, first=False, last=False):
            wait_pre(i, p)
            if not first:
                wait_scatter(i - 2, p)
            issue_gather(i, p)
            issue_idxd(i, p)
            q = 1 - p
            wait_gather(i - 1, q)
            process(i - 1, q)
            if not last:
                issue_pre(i + 1, q)

        issue_pre(0, 0)
        wait_pre(0, 0)
        issue_gather(0, 0)
        issue_idxd(0, 0)
        issue_pre(1, 1)
        iteration(1, 1, first=True)
        iteration(2, 0)

        def pair(gg, carry):
            i = 3 + 2 * gg
            iteration(i, 1)
            iteration(i + 1, 0)
            return carry
        lax.fori_loop(0, (n_chunks - 4) // 2, pair, 0)

        iteration(n_chunks - 1, 1, last=True)
        wait_gather(n_chunks - 1, 1)
        process(n_chunks - 1, 1)
        wait_scatter(n_chunks - 2, 0)
        wait_scatter(n_chunks - 1, 1)

        plsc.subcore_barrier()
        _dump_shared(acc, out_hbm, s, c, n)
        plsc.subcore_barrier()


def _pass_b(hat, vt, c2, src, dst, n):
    m, d = hat.shape
    n_chunks = m // (NC * NS * CH)
    assert n_chunks % 2 == 0 and n_chunks >= 6
    mesh = plsc.VectorSubcoreMesh(core_axis_name="c", subcore_axis_name="s",
                                  num_cores=NC, num_subcores=NS)
    out_type = [
        jax.ShapeDtypeStruct((NC * n, d), f32),
        jax.ShapeDtypeStruct((NC * n, d), f32),
    ]
    buf = [
        pltpu.VMEM((CH,), jnp.int32),
        pltpu.VMEM((CH,), jnp.int32),
        pltpu.VMEM((CH, d), f32),
        pltpu.VMEM((CH, d), f32),
    ]
    scratch = buf + buf + [
        pltpu.VMEM((ZB, d), f32),
        pltpu.VMEM_SHARED((n, d), f32),
    ] + [pltpu.SemaphoreType.DMA] * 8
    kern = pl.kernel(functools.partial(_pass_b_body, n_chunks),
                     out_type=out_type, mesh=mesh, scratch_types=scratch)
    return kern(hat, vt, c2, src, dst)


def _final_body(h_ref, p_ref, a1_ref, ssp_ref, svp_ref, spp_ref,
                wc1_ref, bc1_ref, hout_ref, pout_ref):
    denom = ssp_ref[0] + ssp_ref[1] + 1e-6
    sv = (svp_ref[0] + svp_ref[1]) / denom
    sp = (spp_ref[0] + spp_ref[1]) / denom
    h = h_ref[...]
    p = p_ref[...]
    h_new = a1_ref[...] + sv
    p_new = _dot(p, wc1_ref[...]) + bc1_ref[...] + sp
    hout_ref[...] = h + jnp.maximum(h_new, 0.0)
    pout_ref[...] = p + jnp.tanh(p_new)


def _final(h, p, a1, ssp, svp, spp, WC1, bC1):
    n, d = h.shape
    bn = 512
    row_spec = pl.BlockSpec((bn, d), lambda i: (i, 0))
    part_spec = pl.BlockSpec((NC, bn, d), lambda i: (0, i, 0))
    return pl.pallas_call(
        _final_body,
        grid=(pl.cdiv(n, bn),),
        in_specs=[row_spec, row_spec, row_spec, part_spec, part_spec,
                  part_spec,
                  pl.BlockSpec((d, d), lambda i: (0, 0)),
                  pl.BlockSpec((1, d), lambda i: (0, 0))],
        out_specs=[row_spec, row_spec],
        out_shape=[jax.ShapeDtypeStruct((n, d), f32),
                   jax.ShapeDtypeStruct((n, d), f32)],
    )(h, p, a1, ssp.reshape(NC, n, d), svp.reshape(NC, n, d),
      spp.reshape(NC, n, d), WC1, bC1.reshape(1, d))


def kernel(h, e, p, WA1, bA1, WA2, bA2, WB1, bB1, WB2, bB2, WB3, bB3,
           WC1, bC1, WC2, bC2, edge_index):
    n, d = h.shape
    src = edge_index[0]
    dst = edge_index[1]

    a1, vt, b1, b2, c2 = _node_dense(
        h, p, WA1, bA1, WA2, bA2, WB1, bB1, WB2, bB2, WC2, bC2)
    b3 = _edge_dense(e, WB3, bB3)

    sig, ssp = _pass_a(b3, b1, b2, src, dst, n)
    svp, spp = _pass_b(sig, vt, c2, src, dst, n)
    e_out = _edge_out(e, sig)
    h_out, p_out = _final(h, p, a1, ssp, svp, spp, WC1, bC1)
    return (h_out, e_out, p_out)


# speedup vs baseline: 1.1589x; 1.0158x over previous
import functools

import jax
import jax.numpy as jnp
from jax import lax
from jax.experimental import pallas as pl
from jax.experimental.pallas import tpu as pltpu
from jax.experimental.pallas import tpu_sc as plsc

f32 = jnp.float32
NC = 2
NS = 16
CH = 40
ZB = 104

_HIGH = lax.Precision.HIGHEST


def _dot(a, b, precision=_HIGH):
    return lax.dot_general(a, b, (((1,), (0,)), ((), ())),
                           precision=precision, preferred_element_type=f32)


def _node_dense_b_body(h_ref, wb1, bb1, wb2, bb2, b1_ref, b2_ref):
    h = h_ref[...]
    b1_ref[...] = _dot(h, wb1[...]) + bb1[...]
    b2_ref[...] = _dot(h, wb2[...]) + bb2[...]


def _node_dense_b(h, WB1, bB1, WB2, bB2):
    n, d = h.shape
    bn = 512
    row_spec = pl.BlockSpec((bn, d), lambda i: (i, 0))
    w_spec = pl.BlockSpec((d, d), lambda i: (0, 0))
    b_spec = pl.BlockSpec((1, d), lambda i: (0, 0))
    return pl.pallas_call(
        _node_dense_b_body,
        grid=(pl.cdiv(n, bn),),
        in_specs=[row_spec] + [w_spec, b_spec] * 2,
        out_specs=[row_spec] * 2,
        out_shape=[jax.ShapeDtypeStruct((n, d), f32)] * 2,
    )(h, WB1, bB1.reshape(1, d), WB2, bB2.reshape(1, d))


def _node_dense_rest_body(h_ref, p_ref, wa1h, wa1p, ba1, wa2h, wa2p, ba2,
                          wc2, bc2, a1_ref, vt_ref, c2_ref):
    h = h_ref[...]
    p = p_ref[...]
    a1_ref[...] = _dot(h, wa1h[...]) + _dot(p, wa1p[...]) + ba1[...]
    vt_ref[...] = _dot(h, wa2h[...]) + _dot(p, wa2p[...]) + ba2[...]
    c2_ref[...] = _dot(p, wc2[...]) + bc2[...]


def _node_dense_rest(h, p, WA1, bA1, WA2, bA2, WC2, bC2):
    n, d = h.shape
    bn = 512
    row_spec = pl.BlockSpec((bn, d), lambda i: (i, 0))
    w_spec = pl.BlockSpec((d, d), lambda i: (0, 0))
    b_spec = pl.BlockSpec((1, d), lambda i: (0, 0))
    return pl.pallas_call(
        _node_dense_rest_body,
        grid=(pl.cdiv(n, bn),),
        in_specs=[row_spec, row_spec] + [w_spec, w_spec, b_spec] * 2
                 + [w_spec, b_spec],
        out_specs=[row_spec] * 3,
        out_shape=[jax.ShapeDtypeStruct((n, d), f32)] * 3,
    )(h, p, WA1[:d], WA1[d:], bA1.reshape(1, d), WA2[:d], WA2[d:],
      bA2.reshape(1, d), WC2, bC2.reshape(1, d))


def _edge_dense_body(e_ref, w_ref, b_ref, out_ref):
    e = e_ref[...]
    w = w_ref[...]
    bf16 = jnp.bfloat16
    eh = e.astype(bf16)
    el = (e - eh.astype(f32)).astype(bf16)
    wh = w.astype(bf16)
    wl = (w - wh.astype(f32)).astype(bf16)
    dflt = lax.Precision.DEFAULT
    out_ref[...] = (_dot(eh, wh, dflt) + _dot(eh, wl, dflt)
                    + _dot(el, wh, dflt) + b_ref[...])


def _edge_out_body(e_ref, sig_ref, out_ref):
    out_ref[...] = e_ref[...] + jnp.maximum(sig_ref[...], 0.0)


def _edge_out(e, hat):
    m, d = e.shape
    bm = 2048
    spec = pl.BlockSpec((bm, d), lambda i: (i, 0))
    return pl.pallas_call(
        _edge_out_body,
        grid=(pl.cdiv(m, bm),),
        in_specs=[spec, spec],
        out_specs=spec,
        out_shape=jax.ShapeDtypeStruct((m, d), f32),
    )(e, hat)


def _edge_dense(e, WB3, bB3):
    m, d = e.shape
    bm = 8192
    return pl.pallas_call(
        _edge_dense_body,
        grid=(pl.cdiv(m, bm),),
        in_specs=[pl.BlockSpec((bm, d), lambda i: (i, 0)),
                  pl.BlockSpec((d, d), lambda i: (0, 0)),
                  pl.BlockSpec((1, d), lambda i: (0, 0))],
        out_specs=pl.BlockSpec((bm, d), lambda i: (i, 0)),
        out_shape=jax.ShapeDtypeStruct((m, d), f32),
    )(e, WB3, bB3.reshape(1, d))


def _stripe(n):
    stripe = (n // NS) // 8 * 8
    tail = n - stripe * NS
    assert stripe % ZB == 0 and tail % 8 == 0 and tail < ZB
    return stripe, tail


def _fill_zb(zb):
    def zloop(j, carry):
        for k in range(zb.shape[1] // 16):
            zb[j, pl.ds(k * 16, 16)] = jnp.zeros((16,), f32)
        return carry
    lax.fori_loop(0, zb.shape[0], zloop, 0)


def _zero_shared(zb, acc, s, n):
    stripe, tail = _stripe(n)
    for q in range(stripe // ZB):
        pltpu.sync_copy(zb, acc.at[pl.ds(s * stripe + q * ZB, ZB)])
    if tail:
        @pl.when(s == 0)
        def _():
            pltpu.sync_copy(zb.at[pl.ds(0, tail)],
                            acc.at[pl.ds(NS * stripe, tail)])


def _dump_shared(acc, out, s, c, n):
    stripe, tail = _stripe(n)
    for q in range(stripe // ZB):
        r = s * stripe + q * ZB
        pltpu.sync_copy(acc.at[pl.ds(r, ZB)], out.at[pl.ds(c * n + r, ZB)])
    if tail:
        @pl.when(s == 0)
        def _():
            pltpu.sync_copy(acc.at[pl.ds(NS * stripe, tail)],
                            out.at[pl.ds(c * n + NS * stripe, tail)])


def _pass_a_body(n_chunks,
                 b3_hbm, b1_hbm, b2_hbm, ei_hbm,
                 hat_hbm, ssp_hbm,
                 idx_s0, idx_d0, b1g0, b2g0, b3v0,
                 idx_s1, idx_d1, b1g1, b2g1, b3v1,
                 sg, zb, acc,
                 sem_i0, sem_i1, sem_n0, sem_n1, sem_o0, sem_o1):
    c = lax.axis_index("c")
    s = lax.axis_index("s")
    n = acc.shape[0]
    _fill_zb(zb)
    _zero_shared(zb, acc, s, n)
    plsc.subcore_barrier()

    tile = c * NS + s
    base = tile * (n_chunks * CH)
    m = NC * NS * n_chunks * CH
    idx_s = (idx_s0, idx_s1)
    idx_d = (idx_d0, idx_d1)
    b1g = (b1g0, b1g1)
    b2g = (b2g0, b2g1)
    b3v = (b3v0, b3v1)
    sem_i = (sem_i0, sem_i1)
    sem_n = (sem_n0, sem_n1)
    sem_o = (sem_o0, sem_o1)

    def issue_idx(i, p):
        eb = base + i * CH
        pltpu.async_copy(ei_hbm.at[pl.ds(eb, CH)], idx_s[p], sem_i[p])
        pltpu.async_copy(ei_hbm.at[pl.ds(m + eb, CH)], idx_d[p], sem_i[p])

    def wait_idx(i, p):
        eb = base + i * CH
        pltpu.make_async_copy(ei_hbm.at[pl.ds(eb, CH)], idx_s[p], sem_i[p]).wait()
        pltpu.make_async_copy(ei_hbm.at[pl.ds(m + eb, CH)], idx_d[p], sem_i[p]).wait()

    def issue_inputs(i, p):
        eb = base + i * CH
        pltpu.async_copy(b1_hbm.at[idx_s[p]], b1g[p], sem_n[p])
        pltpu.async_copy(b2_hbm.at[idx_d[p]], b2g[p], sem_n[p])
        pltpu.async_copy(b3_hbm.at[pl.ds(eb, CH)], b3v[p], sem_n[p])

    def wait_inputs(i, p):
        eb = base + i * CH
        pltpu.make_async_copy(b1_hbm.at[idx_s[p]], b1g[p], sem_n[p]).wait()
        pltpu.make_async_copy(b2_hbm.at[idx_d[p]], b2g[p], sem_n[p]).wait()
        pltpu.make_async_copy(b3_hbm.at[pl.ds(eb, CH)], b3v[p], sem_n[p]).wait()

    def process(i, p):
        def comp(j, inner):
            for k in range(8):
                sl = pl.ds(k * 16, 16)
                hat = b1g[p][j, sl] + b2g[p][j, sl] + b3v[p][j, sl]
                b3v[p][j, sl] = hat
                sg[j, sl] = 1.0 / (1.0 + jnp.exp(-hat))
            return inner
        lax.fori_loop(0, CH, comp, 0)
        pltpu.sync_copy(sg, acc.at[idx_d[p]], add=True)

    def issue_outputs(i, p):
        eb = base + i * CH
        pltpu.async_copy(b3v[p], hat_hbm.at[pl.ds(eb, CH)], sem_o[p])

    def wait_outputs(i, p):
        eb = base + i * CH
        pltpu.make_async_copy(b3v[p], hat_hbm.at[pl.ds(eb, CH)], sem_o[p]).wait()

    def iteration(i, p, first=False, last=False):
        if not first:
            wait_outputs(i - 2, p)
        wait_idx(i, p)
        issue_inputs(i, p)
        q = 1 - p
        wait_inputs(i - 1, q)
        process(i - 1, q)
        if not last:
            issue_idx(i + 1, q)
        issue_outputs(i - 1, q)

    issue_idx(0, 0)
    wait_idx(0, 0)
    issue_inputs(0, 0)
    issue_idx(1, 1)
    iteration(1, 1, first=True)
    iteration(2, 0)

    def pair(gg, carry):
        i = 3 + 2 * gg
        iteration(i, 1)
        iteration(i + 1, 0)
        return carry
    lax.fori_loop(0, (n_chunks - 4) // 2, pair, 0)

    iteration(n_chunks - 1, 1, last=True)
    wait_outputs(n_chunks - 2, 0)
    wait_inputs(n_chunks - 1, 1)
    process(n_chunks - 1, 1)
    issue_outputs(n_chunks - 1, 1)
    wait_outputs(n_chunks - 1, 1)

    plsc.subcore_barrier()
    _dump_shared(acc, ssp_hbm, s, c, n)


def _pass_a(b3, b1, b2, ei, n):
    m, d = b3.shape
    n_chunks = m // (NC * NS * CH)
    assert n_chunks % 2 == 0 and n_chunks >= 6
    mesh = plsc.VectorSubcoreMesh(core_axis_name="c", subcore_axis_name="s",
                                  num_cores=NC, num_subcores=NS)
    out_type = [
        jax.ShapeDtypeStruct((m, d), f32),
        jax.ShapeDtypeStruct((NC * n, d), f32),
    ]
    buf = [
        pltpu.VMEM((CH,), jnp.int32),
        pltpu.VMEM((CH,), jnp.int32),
        pltpu.VMEM((CH, d), f32),
        pltpu.VMEM((CH, d), f32),
        pltpu.VMEM((CH, d), f32),
    ]
    scratch = buf + buf + [
        pltpu.VMEM((CH, d), f32),
        pltpu.VMEM((ZB, d), f32),
        pltpu.VMEM_SHARED((n, d), f32),
    ] + [pltpu.SemaphoreType.DMA] * 6
    kern = pl.kernel(functools.partial(_pass_a_body, n_chunks),
                     out_type=out_type, mesh=mesh, scratch_types=scratch)
    return kern(b3, b1, b2, ei)


def _pass_b_body(n_chunks,
                 hat_hbm, vt_hbm, c2_hbm, ei_hbm,
                 svp_hbm, spp_hbm,
                 idx_s0, idx_d0, tg0, sv0,
                 idx_s1, idx_d1, tg1, sv1,
                 zb, acc, sem_p0, sem_p1, sem_g0, sem_g1,
                 sem_d0, sem_d1, sem_s0, sem_s1):
    c = lax.axis_index("c")
    s = lax.axis_index("s")
    n = acc.shape[0]
    tile = c * NS + s
    base = tile * (n_chunks * CH)
    m = NC * NS * n_chunks * CH
    _fill_zb(zb)
    idx_s = (idx_s0, idx_s1)
    idx_d = (idx_d0, idx_d1)
    tg = (tg0, tg1)
    sv = (sv0, sv1)
    sem_p = (sem_p0, sem_p1)
    sem_g = (sem_g0, sem_g1)
    sem_d = (sem_d0, sem_d1)
    sem_s = (sem_s0, sem_s1)

    for tab_hbm, out_hbm in ((vt_hbm, svp_hbm), (c2_hbm, spp_hbm)):
        _zero_shared(zb, acc, s, n)
        plsc.subcore_barrier()

        def issue_pre(i, p):
            eb = base + i * CH
            pltpu.async_copy(ei_hbm.at[pl.ds(eb, CH)], idx_s[p], sem_p[p])
            pltpu.async_copy(hat_hbm.at[pl.ds(eb, CH)], sv[p], sem_p[p])

        def wait_pre(i, p):
            eb = base + i * CH
            pltpu.make_async_copy(ei_hbm.at[pl.ds(eb, CH)], idx_s[p], sem_p[p]).wait()
            pltpu.make_async_copy(hat_hbm.at[pl.ds(eb, CH)], sv[p], sem_p[p]).wait()

        def issue_idxd(i, p):
            eb = base + i * CH
            pltpu.async_copy(ei_hbm.at[pl.ds(m + eb, CH)], idx_d[p], sem_d[p])

        def wait_idxd(i, p):
            eb = base + i * CH
            pltpu.make_async_copy(ei_hbm.at[pl.ds(m + eb, CH)], idx_d[p], sem_d[p]).wait()

        def issue_gather(i, p):
            pltpu.async_copy(tab_hbm.at[idx_s[p]], tg[p], sem_g[p])

        def wait_gather(i, p):
            pltpu.make_async_copy(tab_hbm.at[idx_s[p]], tg[p], sem_g[p]).wait()

        def wait_scatter(i, p):
            pltpu.make_async_copy(tg[p], acc.at[idx_d[p]], sem_s[p]).wait()

        def process(i, p):
            def comp(j, inner):
                for k in range(8):
                    sl = pl.ds(k * 16, 16)
                    sig = 1.0 / (1.0 + jnp.exp(-sv[p][j, sl]))
                    tg[p][j, sl] = sig * tg[p][j, sl]
                return inner
            lax.fori_loop(0, CH, comp, 0)
            wait_idxd(i, p)
            pltpu.async_copy(tg[p], acc.at[idx_d[p]], sem_s[p], add=True)

        def iteration(i, p, first=False, last=False):
            wait_pre(i, p)
            if not first:
                wait_scatter(i - 2, p)
            issue_gather(i, p)
            issue_idxd(i, p)
            q = 1 - p
            wait_gather(i - 1, q)
            process(i - 1, q)
            if not last:
                issue_pre(i + 1, q)

        issue_pre(0, 0)
        wait_pre(0, 0)
        issue_gather(0, 0)
        issue_idxd(0, 0)
        issue_pre(1, 1)
        iteration(1, 1, first=True)
        iteration(2, 0)

        def pair(gg, carry):
            i = 3 + 2 * gg
            iteration(i, 1)
            iteration(i + 1, 0)
            return carry
        lax.fori_loop(0, (n_chunks - 4) // 2, pair, 0)

        iteration(n_chunks - 1, 1, last=True)
        wait_gather(n_chunks - 1, 1)
        process(n_chunks - 1, 1)
        wait_scatter(n_chunks - 2, 0)
        wait_scatter(n_chunks - 1, 1)

        plsc.subcore_barrier()
        _dump_shared(acc, out_hbm, s, c, n)
        plsc.subcore_barrier()


def _pass_b(hat, vt, c2, ei, n):
    m, d = hat.shape
    n_chunks = m // (NC * NS * CH)
    assert n_chunks % 2 == 0 and n_chunks >= 6
    mesh = plsc.VectorSubcoreMesh(core_axis_name="c", subcore_axis_name="s",
                                  num_cores=NC, num_subcores=NS)
    out_type = [
        jax.ShapeDtypeStruct((NC * n, d), f32),
        jax.ShapeDtypeStruct((NC * n, d), f32),
    ]
    buf = [
        pltpu.VMEM((CH,), jnp.int32),
        pltpu.VMEM((CH,), jnp.int32),
        pltpu.VMEM((CH, d), f32),
        pltpu.VMEM((CH, d), f32),
    ]
    scratch = buf + buf + [
        pltpu.VMEM((ZB, d), f32),
        pltpu.VMEM_SHARED((n, d), f32),
    ] + [pltpu.SemaphoreType.DMA] * 8
    kern = pl.kernel(functools.partial(_pass_b_body, n_chunks),
                     out_type=out_type, mesh=mesh, scratch_types=scratch)
    return kern(hat, vt, c2, ei)


def _final_body(h_ref, p_ref, a1_ref, ssp_ref, svp_ref, spp_ref,
                wc1_ref, bc1_ref, hout_ref, pout_ref):
    denom = ssp_ref[0] + ssp_ref[1] + 1e-6
    sv = (svp_ref[0] + svp_ref[1]) / denom
    sp = (spp_ref[0] + spp_ref[1]) / denom
    h = h_ref[...]
    p = p_ref[...]
    h_new = a1_ref[...] + sv
    p_new = _dot(p, wc1_ref[...]) + bc1_ref[...] + sp
    hout_ref[...] = h + jnp.maximum(h_new, 0.0)
    pout_ref[...] = p + jnp.tanh(p_new)


def _final(h, p, a1, ssp, svp, spp, WC1, bC1):
    n, d = h.shape
    bn = 512
    row_spec = pl.BlockSpec((bn, d), lambda i: (i, 0))
    part_spec = pl.BlockSpec((NC, bn, d), lambda i: (0, i, 0))
    return pl.pallas_call(
        _final_body,
        grid=(pl.cdiv(n, bn),),
        in_specs=[row_spec, row_spec, row_spec, part_spec, part_spec,
                  part_spec,
                  pl.BlockSpec((d, d), lambda i: (0, 0)),
                  pl.BlockSpec((1, d), lambda i: (0, 0))],
        out_specs=[row_spec, row_spec],
        out_shape=[jax.ShapeDtypeStruct((n, d), f32),
                   jax.ShapeDtypeStruct((n, d), f32)],
    )(h, p, a1, ssp.reshape(NC, n, d), svp.reshape(NC, n, d),
      spp.reshape(NC, n, d), WC1, bC1.reshape(1, d))


def kernel(h, e, p, WA1, bA1, WA2, bA2, WB1, bB1, WB2, bB2, WB3, bB3,
           WC1, bC1, WC2, bC2, edge_index):
    n, d = h.shape
    ei = edge_index.reshape(-1)

    b1, b2 = _node_dense_b(h, WB1, bB1, WB2, bB2)
    b3 = _edge_dense(e, WB3, bB3)

    hat, ssp = _pass_a(b3, b1, b2, ei, n)
    a1, vt, c2 = _node_dense_rest(h, p, WA1, bA1, WA2, bA2, WC2, bC2)
    svp, spp = _pass_b(hat, vt, c2, ei, n)
    e_out = _edge_out(e, hat)
    h_out, p_out = _final(h, p, a1, ssp, svp, spp, WC1, bC1)
    return (h_out, e_out, p_out)
